# baseline (speedup 1.0000x reference)
import functools

import jax
import jax.numpy as jnp
from jax import lax
from jax.experimental import pallas as pl
from jax.experimental.pallas import tpu as pltpu
from jax.experimental.pallas import tpu_sc as plsc

B = 16384
F = 26
V = 100000
D = 16
DIN = 13
H = 64
DEEP = 64

TOT = B * F
NW = 32
CHUNK = 256
CPW = TOT // (NW * CHUNK)


def _sc_gather(cat_c, tab_w, fs):
    mesh = plsc.VectorSubcoreMesh(core_axis_name="c", subcore_axis_name="s")
    cpw = B * fs // (NW * CHUNK)

    @functools.partial(
        pl.kernel,
        mesh=mesh,
        compiler_params=pltpu.CompilerParams(use_tc_tiling_on_sc=False),
        out_type=jax.ShapeDtypeStruct((B * fs * D,), jnp.float32),
        scratch_types=[
            pltpu.VMEM((cpw, CHUNK), jnp.int32),
            pltpu.VMEM((CHUNK * D // 128, 128), jnp.int32),
            pltpu.VMEM((CHUNK * D // 128, 128), jnp.int32),
            pltpu.VMEM((CHUNK * D,), jnp.float32),
            pltpu.VMEM((CHUNK * D,), jnp.float32),
            pltpu.SemaphoreType.DMA,
            pltpu.SemaphoreType.DMA,
        ],
    )
    def k(cat_hbm, tab_hbm, out_hbm, idx_v, ga, gb, oa, ob, ma, mb):
        wid = lax.axis_index("s") * 2 + lax.axis_index("c")
        cbase = wid * cpw
        wbase = wid * cpw * CHUNK * D

        pltpu.sync_copy(cat_hbm.at[pl.ds(cbase, cpw)], idx_v)

        lane = lax.iota(jnp.int32, 16)
        dword = lane * V

        def build(r, gbuf):
            e0 = (cbase + r) * CHUNK

            def group(q, _):
                vv = idx_v[r, pl.ds(q * 16, 16)]
                fv = lax.rem(e0 + q * 16 + lane, fs)
                base = fv * (D * V) + vv
                for il in range(16):
                    b = jnp.take(base, jnp.full((16,), il, jnp.int32))
                    gbuf[2 * q + il // 8, pl.ds((il % 8) * 16, 16)] = b + dword
                return 0

            lax.fori_loop(0, CHUNK // 16, group, 0)

        def fire(gbuf, obuf, sem):
            def go(j, _):
                pltpu.make_async_copy(
                    tab_hbm.at[gbuf.at[j]],
                    obuf.at[pl.ds(j * 128, 128)],
                    sem,
                ).start()
                return 0

            lax.fori_loop(0, CHUNK * D // 128, go, 0)

        def drain(gbuf, obuf, sem):
            def go(j, _):
                pltpu.make_async_copy(
                    tab_hbm.at[gbuf.at[j]],
                    obuf.at[pl.ds(j * 128, 128)],
                    sem,
                ).wait()
                return 0

            lax.fori_loop(0, CHUNK * D // 128, go, 0)

        def write(r, obuf):
            pltpu.sync_copy(
                obuf, out_hbm.at[pl.ds(wbase + r * CHUNK * D, CHUNK * D)])

        build(0, ga)
        fire(ga, oa, ma)
        NP = cpw // 2

        def pair(p, _):
            r0 = 2 * p

            build(r0 + 1, gb)
            fire(gb, ob, mb)
            drain(ga, oa, ma)
            write(r0, oa)

            @pl.when(p + 1 < NP)
            def _():
                build(r0 + 2, ga)
                fire(ga, oa, ma)

            drain(gb, ob, mb)
            write(r0 + 1, ob)
            return 0

        lax.fori_loop(0, NP, pair, 0)

    return k(cat_c, tab_w)


def _tc_dense(cf, emb2, emb2b, W1, b1, W2, b2, Wd_de, Wd_emb, Wd_embb, bd,
              Wl_de, Wl_dp, wl_fm, bl):
    BLK = 2048
    grid = (B // BLK,)

    def body(cf_ref, emb_ref, embb_ref, w1_ref, b1_ref, w2_ref, b2_ref,
             wde_ref, wdem_ref, wdemb_ref, bd_ref, wl1_ref, wl2_ref,
             wlf_ref, bl_ref, out_ref):
        cf_blk = cf_ref[...]
        h = jnp.maximum(
            jnp.dot(cf_blk, w1_ref[...], preferred_element_type=jnp.float32)
            + b1_ref[...], 0.0)
        de = jnp.maximum(
            jnp.dot(h, w2_ref[...], preferred_element_type=jnp.float32)
            + b2_ref[...], 0.0)
        emb = emb_ref[...]
        embb = embb_ref[...]
        deep = jnp.maximum(
            jnp.dot(de, wde_ref[...], preferred_element_type=jnp.float32)
            + jnp.dot(emb, wdem_ref[...], preferred_element_type=jnp.float32)
            + jnp.dot(embb, wdemb_ref[...],
                      preferred_element_type=jnp.float32)
            + bd_ref[...], 0.0)
        s1 = (jnp.sum(de, axis=1, keepdims=True)
              + jnp.sum(emb, axis=1, keepdims=True)
              + jnp.sum(embb, axis=1, keepdims=True))
        s2 = (jnp.sum(de * de, axis=1, keepdims=True)
              + jnp.sum(emb * emb, axis=1, keepdims=True)
              + jnp.sum(embb * embb, axis=1, keepdims=True))
        fm = 0.5 * (s1 * s1 - s2)
        out_ref[...] = (
            jnp.dot(de, wl1_ref[...], preferred_element_type=jnp.float32)
            + jnp.dot(deep, wl2_ref[...], preferred_element_type=jnp.float32)
            + fm * wlf_ref[...] + bl_ref[...])

    full = lambda shape: pl.BlockSpec(shape, lambda i: (0,) * len(shape))
    return pl.pallas_call(
        body,
        grid=grid,
        in_specs=[
            pl.BlockSpec((BLK, DIN), lambda i: (i, 0)),
            pl.BlockSpec((BLK, 13 * D), lambda i: (i, 0)),
            pl.BlockSpec((BLK, 13 * D), lambda i: (i, 0)),
            full((DIN, H)),
            full((1, H)),
            full((H, D)),
            full((1, D)),
            full((D, DEEP)),
            full((13 * D, DEEP)),
            full((13 * D, DEEP)),
            full((1, DEEP)),
            full((D, 1)),
            full((DEEP, 1)),
            full((1, 1)),
            full((1, 1)),
        ],
        out_specs=pl.BlockSpec((BLK, 1), lambda i: (i, 0)),
        out_shape=jax.ShapeDtypeStruct((B, 1), jnp.float32),
    )(cf, emb2, emb2b, W1, b1, W2, b2, Wd_de, Wd_emb, Wd_embb, bd,
       Wl_de, Wl_dp, wl_fm, bl)


def kernel(count_features, category_features, tables, W1, b1, W2, b2, Wd, bd, Wl, bl):
    cat32 = category_features.astype(jnp.int32)
    FH = F // 2
    embs = []
    for g in range(2):
        cat_s = cat32[:, g * FH:(g + 1) * FH].reshape(B * FH // CHUNK, CHUNK)
        tab_s = lax.slice_in_dim(tables, g * FH, (g + 1) * FH, axis=0)
        tab_s = tab_s.transpose(0, 2, 1).reshape(FH * D * V)
        embs.append(_sc_gather(cat_s, tab_s, FH).reshape(B, FH * D))
    logits = _tc_dense(
        count_features, embs[0], embs[1], W1, b1.reshape(1, H), W2,
        b2.reshape(1, D), Wd[:D], Wd[D:D + FH * D], Wd[D + FH * D:],
        bd.reshape(1, DEEP),
        Wl[:D], Wl[D:D + DEEP], Wl[D + DEEP:].reshape(1, 1), bl.reshape(1, 1))
    return logits


# speedup vs baseline: 1.0526x; 1.0526x over previous
import functools

import jax
import jax.numpy as jnp
from jax import lax
from jax.experimental import pallas as pl
from jax.experimental.pallas import tpu as pltpu
from jax.experimental.pallas import tpu_sc as plsc

B = 16384
F = 26
V = 100000
D = 16
DIN = 13
H = 64
DEEP = 64

TOT = B * F
NW = 32
CHUNK = 256
CPW = TOT // (NW * CHUNK)


def _sc_gather(cat_c, tab_w):
    mesh = plsc.VectorSubcoreMesh(core_axis_name="c", subcore_axis_name="s")

    @functools.partial(
        pl.kernel,
        mesh=mesh,
        compiler_params=pltpu.CompilerParams(use_tc_tiling_on_sc=False),
        out_type=jax.ShapeDtypeStruct((TOT * D,), jnp.float32),
        scratch_types=[
            pltpu.VMEM((CPW, CHUNK), jnp.int32),
            pltpu.VMEM((CHUNK * D // 128, 128), jnp.int32),
            pltpu.VMEM((CHUNK * D // 128, 128), jnp.int32),
            pltpu.VMEM((CHUNK * D,), jnp.float32),
            pltpu.VMEM((CHUNK * D,), jnp.float32),
            pltpu.SemaphoreType.DMA,
            pltpu.SemaphoreType.DMA,
        ],
    )
    def k(cat_hbm, tab_hbm, out_hbm, idx_v, ga, gb, oa, ob, ma, mb):
        wid = lax.axis_index("s") * 2 + lax.axis_index("c")
        cbase = wid * CPW
        wbase = wid * CPW * CHUNK * D

        pltpu.sync_copy(cat_hbm.at[pl.ds(cbase, CPW)], idx_v)

        lane = lax.iota(jnp.int32, 16)
        dword = lane * V

        def build(r, gbuf):
            e0 = (cbase + r) * CHUNK

            def group(q, _):
                vv = idx_v[r, pl.ds(q * 16, 16)]
                fv = lax.rem(e0 + q * 16 + lane, F)
                base = fv * (D * V) + vv
                for il in range(16):
                    b = jnp.take(base, jnp.full((16,), il, jnp.int32))
                    gbuf[2 * q + il // 8, pl.ds((il % 8) * 16, 16)] = b + dword
                return 0

            lax.fori_loop(0, CHUNK // 16, group, 0)

        def fire(gbuf, obuf, sem):
            def go(j, _):
                pltpu.make_async_copy(
                    tab_hbm.at[gbuf.at[j]],
                    obuf.at[pl.ds(j * 128, 128)],
                    sem,
                ).start()
                return 0

            lax.fori_loop(0, CHUNK * D // 128, go, 0)

        def drain(gbuf, obuf, sem):
            def go(j, _):
                pltpu.make_async_copy(
                    tab_hbm.at[gbuf.at[j]],
                    obuf.at[pl.ds(j * 128, 128)],
                    sem,
                ).wait()
                return 0

            lax.fori_loop(0, CHUNK * D // 128, go, 0)

        def write(r, obuf):
            pltpu.sync_copy(
                obuf, out_hbm.at[pl.ds(wbase + r * CHUNK * D, CHUNK * D)])

        build(0, ga)
        fire(ga, oa, ma)
        NP = CPW // 2

        def pair(p, _):
            r0 = 2 * p

            build(r0 + 1, gb)
            fire(gb, ob, mb)
            drain(ga, oa, ma)
            write(r0, oa)

            @pl.when(p + 1 < NP)
            def _():
                build(r0 + 2, ga)
                fire(ga, oa, ma)

            drain(gb, ob, mb)
            write(r0 + 1, ob)
            return 0

        lax.fori_loop(0, NP, pair, 0)

    return k(cat_c, tab_w)


def _tc_dense(cf, emb2, W1, b1, W2, b2, Wd_de, Wd_emb, bd, Wl_de, Wl_dp, wl_fm, bl):
    BLK = 2048
    grid = (B // BLK,)

    def body(cf_ref, emb_ref, w1_ref, b1_ref, w2_ref, b2_ref, wde_ref,
             wdem_ref, bd_ref, wl1_ref, wl2_ref, wlf_ref, bl_ref, out_ref):
        cf_blk = cf_ref[...]
        h = jnp.maximum(
            jnp.dot(cf_blk, w1_ref[...], preferred_element_type=jnp.float32)
            + b1_ref[...], 0.0)
        de = jnp.maximum(
            jnp.dot(h, w2_ref[...], preferred_element_type=jnp.float32)
            + b2_ref[...], 0.0)
        emb = emb_ref[...]
        deep = jnp.maximum(
            jnp.dot(de, wde_ref[...], preferred_element_type=jnp.float32)
            + jnp.dot(emb, wdem_ref[...], preferred_element_type=jnp.float32)
            + bd_ref[...], 0.0)
        s1 = (jnp.sum(de, axis=1, keepdims=True)
              + jnp.sum(emb, axis=1, keepdims=True))
        s2 = (jnp.sum(de * de, axis=1, keepdims=True)
              + jnp.sum(emb * emb, axis=1, keepdims=True))
        fm = 0.5 * (s1 * s1 - s2)
        out_ref[...] = (
            jnp.dot(de, wl1_ref[...], preferred_element_type=jnp.float32)
            + jnp.dot(deep, wl2_ref[...], preferred_element_type=jnp.float32)
            + fm * wlf_ref[...] + bl_ref[...])

    full = lambda shape: pl.BlockSpec(shape, lambda i: (0,) * len(shape))
    return pl.pallas_call(
        body,
        grid=grid,
        in_specs=[
            pl.BlockSpec((BLK, DIN), lambda i: (i, 0)),
            pl.BlockSpec((BLK, F * D), lambda i: (i, 0)),
            full((DIN, H)),
            full((1, H)),
            full((H, D)),
            full((1, D)),
            full((D, DEEP)),
            full((F * D, DEEP)),
            full((1, DEEP)),
            full((D, 1)),
            full((DEEP, 1)),
            full((1, 1)),
            full((1, 1)),
        ],
        out_specs=pl.BlockSpec((BLK, 1), lambda i: (i, 0)),
        out_shape=jax.ShapeDtypeStruct((B, 1), jnp.float32),
    )(cf, emb2, W1, b1, W2, b2, Wd_de, Wd_emb, bd, Wl_de, Wl_dp, wl_fm, bl)


def kernel(count_features, category_features, tables, W1, b1, W2, b2, Wd, bd, Wl, bl):
    cat_c = category_features.astype(jnp.int32).reshape(TOT // CHUNK, CHUNK)
    tab_w = tables.transpose(0, 2, 1).reshape(F * D * V)
    emb_flat = _sc_gather(cat_c, tab_w)
    emb2 = emb_flat.reshape(B, F * D)
    logits = _tc_dense(
        count_features, emb2, W1, b1.reshape(1, H), W2, b2.reshape(1, D),
        Wd[:D], Wd[D:], bd.reshape(1, DEEP),
        Wl[:D], Wl[D:D + DEEP], Wl[D + DEEP:].reshape(1, 1), bl.reshape(1, 1))
    return logits
